# baseline (speedup 1.0000x reference)
import functools

import jax
import jax.numpy as jnp
from jax import lax
from jax.experimental import pallas as pl
from jax.experimental.pallas import tpu as pltpu
from jax.experimental.pallas import tpu_sc as plsc

VOCAB = 1000000
EMBED = 32
BATCH = 16384
NC, NS, L = 2, 16, 16
NW = NC * NS
BPW = BATCH // NW
CH = 128
NCH = BPW // CH

NGRP = 4
NT = 7813
GRP_WORDS = NT * 1024
TOTAL_TILES = NGRP * NT
TPW = (TOTAL_TILES + NW - 1) // NW
FIRE = 8

_mesh = plsc.VectorSubcoreMesh(core_axis_name="c", subcore_axis_name="s")


@functools.partial(
    pl.kernel,
    out_type=(
        jax.ShapeDtypeStruct((NGRP, NT, 8, 128), jnp.float32),
        jax.ShapeDtypeStruct((NGRP, NT, 8, 128), jnp.float32),
    ),
    mesh=_mesh,
    compiler_params=pltpu.CompilerParams(needs_layout_passes=False),
    scratch_types=[
        pltpu.SemaphoreType.DMA,
    ],
)
def _sc_retile(ct_hbm, ot_hbm, tc_hbm, to_hbm, cf_hbm, of_hbm, sem):
    wid = lax.axis_index("s") * NC + lax.axis_index("c")
    k0 = wid * TPW

    def copy_tile(src, dst, k, fire):
        g = k // NT
        t = k % NT
        sref = src.at[pl.ds(pl.multiple_of(g * 8, 8), 8),
                      pl.ds(pl.multiple_of(t * 128, 128), 128)]
        dref = dst.at[g, t]
        if fire:
            pltpu.async_copy(sref, dref, sem)
        else:
            pltpu.make_async_copy(sref, dref, sem).wait()

    for src, dst in ((ct_hbm, cf_hbm), (ot_hbm, of_hbm)):
        def batch(b, carry):
            for phase in (True, False):
                for i in range(FIRE):
                    k = k0 + b * FIRE + i

                    @pl.when(jnp.logical_and(k < TOTAL_TILES,
                                             k % NT < NT - 1))
                    def _tile():
                        copy_tile(src, dst, k, phase)
            return carry

        lax.fori_loop(0, (TPW + FIRE - 1) // FIRE, batch, 0)

    @pl.when(wid < NGRP)
    def _tails():
        for src, dst in ((tc_hbm, cf_hbm), (to_hbm, of_hbm)):
            pltpu.sync_copy(
                src.at[pl.ds(pl.multiple_of(wid * 8, 8), 8), :],
                dst.at[wid, NT - 1])


@functools.partial(
    pl.kernel,
    out_type=jax.ShapeDtypeStruct((BATCH,), jnp.float32),
    mesh=_mesh,
    compiler_params=pltpu.CompilerParams(
        use_tc_tiling_on_sc=False, needs_layout_passes=False),
    scratch_types=[
        pltpu.VMEM((NCH, CH), jnp.int32),
        pltpu.VMEM((NCH, CH), jnp.int32),
        pltpu.VMEM((EMBED, BPW), jnp.float32),
        pltpu.VMEM((EMBED, BPW), jnp.float32),
        pltpu.VMEM((BPW,), jnp.float32),
        pltpu.SemaphoreType.DMA,
    ],
)
def _sc_dots(x_hbm, y_hbm, cf_hbm, of_hbm, dots_hbm, xp, yp, cbuf, obuf, dv,
             sem):
    wid = lax.axis_index("s") * NC + lax.axis_index("c")
    base = wid * BPW
    pltpu.sync_copy(x_hbm.at[pl.ds(wid * NCH, NCH)], xp)
    pltpu.sync_copy(y_hbm.at[pl.ds(wid * NCH, NCH)], yp)
    for j in range(NCH):
        for k in range(CH // L):
            sl = pl.ds(k * L, L)
            vx = xp[j, sl]
            vy = yp[j, sl]
            xp[j, sl] = lax.shift_left(lax.shift_right_logical(vx, 7), 10) \
                + jnp.bitwise_and(vx, 127)
            yp[j, sl] = lax.shift_left(lax.shift_right_logical(vy, 7), 10) \
                + jnp.bitwise_and(vy, 127)

    for j in range(NCH):
        copies = []
        for c in range(EMBED):
            cbase = (c // 8) * GRP_WORDS + (c % 8) * 128
            clen = (NT - 1) * 1024 + 128
            copies.append(
                pltpu.async_copy(
                    cf_hbm.at[pl.ds(cbase, clen)].at[xp.at[j]],
                    cbuf.at[c, pl.ds(j * CH, CH)], sem))
            copies.append(
                pltpu.async_copy(
                    of_hbm.at[pl.ds(cbase, clen)].at[yp.at[j]],
                    obuf.at[c, pl.ds(j * CH, CH)], sem))
        for cp in copies:
            cp.wait()

    def body(g, carry):
        sl = pl.ds(g * L, L)
        acc = cbuf[0, sl] * obuf[0, sl]
        for c in range(1, EMBED):
            acc = acc + cbuf[c, sl] * obuf[c, sl]
        dv[sl] = acc
        return carry

    lax.fori_loop(0, BPW // L, body, 0)
    pltpu.sync_copy(dv, dots_hbm.at[pl.ds(base, BPW)])


def _tc_loss_body(d_ref, o_ref):
    d = d_ref[...]
    neg_abs = -jnp.abs(d)
    ls = jnp.minimum(d, 0.0) - jnp.log(1.0 + jnp.exp(neg_abs))
    o_ref[0, 0] = -jnp.sum(ls) / BATCH


_tc_loss = pl.pallas_call(
    _tc_loss_body,
    out_shape=jax.ShapeDtypeStruct((1, 1), jnp.float32),
    out_specs=pl.BlockSpec(memory_space=pltpu.SMEM),
)


def kernel(x, y, center_weight, out_weight):
    ct = center_weight.T
    ot = out_weight.T
    tpad = ((0, 0), (0, 128 - (VOCAB - (NT - 1) * 128)))
    tc = jnp.pad(center_weight[(NT - 1) * 128:].T, tpad)
    to = jnp.pad(out_weight[(NT - 1) * 128:].T, tpad)
    cf4, of4 = _sc_retile(ct, ot, tc, to)
    cf = cf4.reshape(NGRP * NT * 8 * 128)
    of = of4.reshape(NGRP * NT * 8 * 128)
    x2 = x.reshape(NW * NCH, CH)
    y2 = y.reshape(NW * NCH, CH)
    dots = _sc_dots(x2, y2, cf, of)
    loss = _tc_loss(dots.reshape(BATCH // 128, 128))
    return loss[0, 0]


# speedup vs baseline: 31.0366x; 31.0366x over previous
import functools

import jax
import jax.numpy as jnp
from jax import lax
from jax.experimental import pallas as pl
from jax.experimental.pallas import tpu as pltpu
from jax.experimental.pallas import tpu_sc as plsc

VOCAB = 1000000
EMBED = 32
BATCH = 16384
NC, NS, L = 2, 16, 16
NW = NC * NS
BPW = BATCH // NW
CH = 128
NCH = BPW // CH

NGRP = 4
NT = 7813
GRP_WORDS = NT * 1024
TOTAL_TILES = NGRP * NT
WIN = 16
NWIN = 61

_mesh = plsc.VectorSubcoreMesh(core_axis_name="c", subcore_axis_name="s")


@functools.partial(
    pl.kernel,
    out_type=(
        jax.ShapeDtypeStruct((NGRP, NT, 8, 128), jnp.float32),
        jax.ShapeDtypeStruct((NGRP, NT, 8, 128), jnp.float32),
    ),
    mesh=_mesh,
    compiler_params=pltpu.CompilerParams(needs_layout_passes=False),
    scratch_types=[
        pltpu.VMEM((WIN, 8, 128), jnp.float32),
        pltpu.VMEM((WIN, 8, 128), jnp.float32),
        pltpu.VMEM((8, 128), jnp.float32),
        pltpu.SemaphoreType.DMA((2,)),
    ],
)
def _sc_retile(ct_hbm, ot_hbm, tc_hbm, to_hbm, cf_hbm, of_hbm, vb0, vb1,
               tbuf, in_sems):
    wid = lax.axis_index("s") * NC + lax.axis_index("c")
    g = wid % NGRP
    s = wid // NGRP
    t_base = s * (NWIN * WIN)
    vbufs = (vb0, vb1)

    def src_tile(src, t):
        return src.at[pl.ds(pl.multiple_of(g * 8, 8), 8),
                      pl.ds(pl.multiple_of(t * 128, 128), 128)]

    for src, dst in ((ct_hbm, cf_hbm), (ot_hbm, of_hbm)):
        def fire_in(w, slot):
            t0 = t_base + w * WIN
            for i in range(WIN):
                pltpu.async_copy(src_tile(src, t0 + i),
                                 vbufs[slot].at[i], in_sems.at[slot])

        def wait_in(w, slot):
            t0 = t_base + w * WIN
            for i in range(WIN):
                pltpu.make_async_copy(src_tile(src, t0 + i),
                                      vbufs[slot].at[i],
                                      in_sems.at[slot]).wait()

        def flush_out(w, slot):
            t0 = t_base + w * WIN
            pltpu.sync_copy(
                vbufs[slot],
                dst.at[g, pl.ds(pl.multiple_of(t0, WIN), WIN)])

        fire_in(0, 0)
        fire_in(1, 1)

        def pair(b, carry):
            for i in range(2):
                w = b * 2 + i
                wait_in(w, i)
                flush_out(w, i)

                @pl.when(w + 2 < NWIN)
                def _next():
                    fire_in(w + 2, i)
            return carry

        lax.fori_loop(0, NWIN // 2, pair, 0)
        wait_in(NWIN - 1, (NWIN - 1) % 2)
        flush_out(NWIN - 1, (NWIN - 1) % 2)

    @pl.when(wid < NGRP)
    def _tails():
        for src, tsrc, dst in ((ct_hbm, tc_hbm, cf_hbm),
                               (ot_hbm, to_hbm, of_hbm)):
            for t in range(8 * NWIN * WIN, NT - 1):
                pltpu.sync_copy(src_tile(src, t), tbuf)
                pltpu.sync_copy(tbuf, dst.at[g, t])
            pltpu.sync_copy(
                tsrc.at[pl.ds(pl.multiple_of(g * 8, 8), 8), :], tbuf)
            pltpu.sync_copy(tbuf, dst.at[g, NT - 1])


@functools.partial(
    pl.kernel,
    out_type=jax.ShapeDtypeStruct((BATCH,), jnp.float32),
    mesh=_mesh,
    compiler_params=pltpu.CompilerParams(
        use_tc_tiling_on_sc=False, needs_layout_passes=False),
    scratch_types=[
        pltpu.VMEM((NCH, CH), jnp.int32),
        pltpu.VMEM((NCH, CH), jnp.int32),
        pltpu.VMEM((EMBED, BPW), jnp.float32),
        pltpu.VMEM((EMBED, BPW), jnp.float32),
        pltpu.VMEM((BPW,), jnp.float32),
        pltpu.SemaphoreType.DMA,
    ],
)
def _sc_dots(x_hbm, y_hbm, cf_hbm, of_hbm, dots_hbm, xp, yp, cbuf, obuf, dv,
             sem):
    wid = lax.axis_index("s") * NC + lax.axis_index("c")
    base = wid * BPW
    pltpu.sync_copy(x_hbm.at[pl.ds(wid * NCH, NCH)], xp)
    pltpu.sync_copy(y_hbm.at[pl.ds(wid * NCH, NCH)], yp)
    for j in range(NCH):
        for k in range(CH // L):
            sl = pl.ds(k * L, L)
            vx = xp[j, sl]
            vy = yp[j, sl]
            xp[j, sl] = lax.shift_left(lax.shift_right_logical(vx, 7), 10) \
                + jnp.bitwise_and(vx, 127)
            yp[j, sl] = lax.shift_left(lax.shift_right_logical(vy, 7), 10) \
                + jnp.bitwise_and(vy, 127)

    for j in range(NCH):
        copies = []
        for c in range(EMBED):
            cbase = (c // 8) * GRP_WORDS + (c % 8) * 128
            clen = (NT - 1) * 1024 + 128
            copies.append(
                pltpu.async_copy(
                    cf_hbm.at[pl.ds(cbase, clen)].at[xp.at[j]],
                    cbuf.at[c, pl.ds(j * CH, CH)], sem))
            copies.append(
                pltpu.async_copy(
                    of_hbm.at[pl.ds(cbase, clen)].at[yp.at[j]],
                    obuf.at[c, pl.ds(j * CH, CH)], sem))
        for cp in copies:
            cp.wait()

    def body(g, carry):
        sl = pl.ds(g * L, L)
        acc = cbuf[0, sl] * obuf[0, sl]
        for c in range(1, EMBED):
            acc = acc + cbuf[c, sl] * obuf[c, sl]
        dv[sl] = acc
        return carry

    lax.fori_loop(0, BPW // L, body, 0)
    pltpu.sync_copy(dv, dots_hbm.at[pl.ds(base, BPW)])


def _tc_loss_body(d_ref, o_ref):
    d = d_ref[...]
    neg_abs = -jnp.abs(d)
    ls = jnp.minimum(d, 0.0) - jnp.log(1.0 + jnp.exp(neg_abs))
    o_ref[0, 0] = -jnp.sum(ls) / BATCH


_tc_loss = pl.pallas_call(
    _tc_loss_body,
    out_shape=jax.ShapeDtypeStruct((1, 1), jnp.float32),
    out_specs=pl.BlockSpec(memory_space=pltpu.SMEM),
)


def kernel(x, y, center_weight, out_weight):
    ct = center_weight.T
    ot = out_weight.T
    tpad = ((0, 0), (0, 128 - (VOCAB - (NT - 1) * 128)))
    tc = jnp.pad(center_weight[(NT - 1) * 128:].T, tpad)
    to = jnp.pad(out_weight[(NT - 1) * 128:].T, tpad)
    cf4, of4 = _sc_retile(ct, ot, tc, to)
    cf = cf4.reshape(NGRP * NT * 8 * 128)
    of = of4.reshape(NGRP * NT * 8 * 128)
    x2 = x.reshape(NW * NCH, CH)
    y2 = y.reshape(NW * NCH, CH)
    dots = _sc_dots(x2, y2, cf, of)
    loss = _tc_loss(dots.reshape(BATCH // 128, 128))
    return loss[0, 0]
